# Initial kernel scaffold; baseline (speedup 1.0000x reference)
import jax
import jax.numpy as jnp
from jax.experimental import pallas as pl


def kernel(x, edge_index, W1, b1, W2, b2):
    raise NotImplementedError("write your pallas kernel here")


# speedup vs baseline: 5.4268x; 5.4268x over previous
import functools

import jax
import jax.numpy as jnp
from jax import lax
from jax.experimental import pallas as pl
from jax.experimental.pallas import tpu as pltpu
from jax.experimental.pallas import tpu_sc as plsc

NC = 2
NS = 16
NW = NC * NS


def _mm1_body(x_ref, w_ref, b_ref, o_ref):
    acc = jnp.dot(x_ref[...], w_ref[...], preferred_element_type=jnp.float32)
    o_ref[...] = jnp.maximum(acc + b_ref[...], 0.0)


def _mm2_body(p_ref, h_ref, w_ref, b_ref, o_ref):
    agg = p_ref[0] + p_ref[1]
    acc = jnp.dot(agg, w_ref[...], preferred_element_type=jnp.float32)
    o_ref[...] = jnp.maximum(acc + b_ref[...], 0.0) + h_ref[...]


def _sc_agg(h, src, dst, pad_n, k, ept):
    n, d = h.shape
    rpt = pad_n // NS
    zr = k
    mesh = plsc.VectorSubcoreMesh(core_axis_name="c", subcore_axis_name="s")

    @functools.partial(
        pl.kernel,
        out_type=jax.ShapeDtypeStruct((NC, pad_n, d), jnp.float32),
        mesh=mesh,
        scratch_types=[
            pltpu.VMEM((k,), jnp.int32),
            pltpu.VMEM((k,), jnp.int32),
            pltpu.VMEM((k, d), jnp.float32),
            pltpu.VMEM((zr, d), jnp.float32),
            pltpu.VMEM_SHARED((pad_n, d), jnp.float32),
            pltpu.SemaphoreType.DMA,
        ],
    )
    def agg_kernel(h_hbm, src_hbm, dst_hbm, out_hbm,
                   src_v, dst_v, rows_v, zbuf, shared, sem):
        cid = lax.axis_index("c")
        sid = lax.axis_index("s")

        @pl.loop(0, zr)
        def _(r):
            @pl.loop(0, d, step=16)
            def _(c0):
                zbuf[r, pl.ds(c0, 16)] = jnp.zeros((16,), jnp.float32)

        row0 = sid * rpt

        @pl.loop(0, rpt, step=zr)
        def _(r):
            pltpu.sync_copy(zbuf, shared.at[pl.ds(row0 + r, zr)])

        plsc.subcore_barrier()

        base = (sid * NC + cid) * ept

        @pl.loop(0, ept, step=k)
        def _(e0):
            pltpu.sync_copy(src_hbm.at[pl.ds(base + e0, k)], src_v)
            pltpu.sync_copy(dst_hbm.at[pl.ds(base + e0, k)], dst_v)
            pltpu.async_copy(h_hbm.at[src_v], rows_v, sem).wait()
            pltpu.sync_copy(rows_v, shared.at[dst_v], add=True)

        plsc.subcore_barrier()

        pltpu.sync_copy(shared.at[pl.ds(row0, rpt)],
                        out_hbm.at[cid, pl.ds(row0, rpt)])

    return agg_kernel(h, src, dst)


def kernel(x, edge_index, W1, b1, W2, b2):
    n, d = x.shape
    e = edge_index.shape[1]

    ept = e // NW
    k = 80
    assert ept % k == 0 and ept % 8 == 0
    pad_n = ((n + 8 * NS - 1) // (8 * NS)) * (8 * NS)

    bn = 1000
    grid = (n // bn,)
    b1r = b1.reshape(1, d)
    b2r = b2.reshape(1, d)

    h = pl.pallas_call(
        _mm1_body,
        grid=grid,
        in_specs=[
            pl.BlockSpec((bn, d), lambda i: (i, 0)),
            pl.BlockSpec((d, d), lambda i: (0, 0)),
            pl.BlockSpec((1, d), lambda i: (0, 0)),
        ],
        out_specs=pl.BlockSpec((bn, d), lambda i: (i, 0)),
        out_shape=jax.ShapeDtypeStruct((n, d), jnp.float32),
    )(x, W1, b1r)

    partials = _sc_agg(h, edge_index[0], edge_index[1], pad_n, k, ept)

    out = pl.pallas_call(
        _mm2_body,
        grid=grid,
        in_specs=[
            pl.BlockSpec((NC, bn, d), lambda i: (0, i, 0)),
            pl.BlockSpec((bn, d), lambda i: (i, 0)),
            pl.BlockSpec((d, d), lambda i: (0, 0)),
            pl.BlockSpec((1, d), lambda i: (0, 0)),
        ],
        out_specs=pl.BlockSpec((bn, d), lambda i: (i, 0)),
        out_shape=jax.ShapeDtypeStruct((n, d), jnp.float32),
    )(partials, h, W2, b2r)

    return out
